# baseline (speedup 1.0000x reference)
import functools
import math

import jax
import jax.numpy as jnp
from jax import lax
from jax.experimental import pallas as pl
from jax.experimental.pallas import tpu as pltpu
from jax.experimental.pallas import tpu_sc as plsc

BATCH = 1024
HIST = 200
DIM = 32
TOTAL = BATCH * HIST
NC, NS, LANES = 2, 16, 16
NW = NC * NS
PER_W = TOTAL // NW
KB = 5
CH = KB * 128
NCH = PER_W // CH

MIN_VAR = 0.02
MAX_VAR = 3.0
_LN2 = math.log(2.0)


def _softplus_clamp16(x):
    y = jnp.exp(x) + 1.0
    bits = lax.bitcast_convert_type(y, jnp.int32)
    e = lax.shift_right_logical(bits, 23)
    m = lax.bitcast_convert_type(
        (bits & 0x007FFFFF) | 0x3F800000, jnp.float32)
    s = (m - 1.0) / (m + 1.0)
    s2 = s * s
    t = 2.0 / 7.0
    t = 2.0 / 5.0 + s2 * t
    t = 2.0 / 3.0 + s2 * t
    t = 2.0 + s2 * t
    log_m = s * t
    log_y = (e.astype(jnp.float32) - 127.0) * _LN2 + log_m
    return jnp.minimum(log_y + MIN_VAR, MAX_VAR)


@functools.partial(
    pl.kernel,
    out_type=(
        jax.ShapeDtypeStruct((TOTAL, DIM), jnp.float32),
        jax.ShapeDtypeStruct((TOTAL, DIM), jnp.float32),
    ),
    mesh=plsc.VectorSubcoreMesh(core_axis_name="c", subcore_axis_name="s"),
    scratch_types=[
        pltpu.VMEM((2, CH), jnp.int32),
        pltpu.VMEM((2, CH, DIM), jnp.float32),
        pltpu.VMEM((2, CH, DIM), jnp.float32),
        pltpu.SemaphoreType.DMA,
        pltpu.SemaphoreType.DMA,
    ],
    compiler_params=pltpu.CompilerParams(use_tc_tiling_on_sc=False),
)
def _gauss_embed(ids_hbm, mu_hbm, lv_hbm, mu_out, var_out,
                 idx_v, mu_v, lv_v, sem0, sem1):
    wid = lax.axis_index("s") * NC + lax.axis_index("c")
    base = wid * PER_W
    sems = (sem0, sem1)

    def start(g):
        slot = g & 1
        off = base + g * CH
        pltpu.sync_copy(ids_hbm.at[pl.ds(off, CH)], idx_v.at[slot])
        handles = []
        for jb in range(KB):
            idx_row = idx_v.at[slot, pl.ds(jb * 128, 128)]
            dst = pl.ds(jb * 128, 128)
            handles.append(pltpu.async_copy(
                mu_hbm.at[idx_row], mu_v.at[slot, dst], sems[slot]))
            handles.append(pltpu.async_copy(
                lv_hbm.at[idx_row], lv_v.at[slot, dst], sems[slot]))
        return handles

    pending = start(0)
    for g in range(NCH):
        nxt = start(g + 1) if g + 1 < NCH else None
        for h in pending:
            h.wait()
        slot = g & 1

        def row_body(j, _):
            for half in range(DIM // LANES):
                sl = pl.ds(half * LANES, LANES)
                lv_v[slot, j, sl] = _softplus_clamp16(lv_v[slot, j, sl])
            return 0

        lax.fori_loop(0, CH, row_body, 0)
        off = base + g * CH
        pltpu.sync_copy(mu_v.at[slot], mu_out.at[pl.ds(off, CH)])
        pltpu.sync_copy(lv_v.at[slot], var_out.at[pl.ds(off, CH)])
        pending = nxt


def kernel(ids, mu_weight, log_var_weight):
    ids_flat = ids.astype(jnp.int32).reshape(TOTAL)
    mu_flat, var_flat = _gauss_embed(ids_flat, mu_weight, log_var_weight)
    return (mu_flat.reshape(BATCH, HIST, DIM),
            var_flat.reshape(BATCH, HIST, DIM))
